# baseline (speedup 1.0000x reference)
import functools

import jax
import jax.numpy as jnp
from jax import lax
from jax.experimental import pallas as pl
from jax.experimental.pallas import tpu as pltpu
from jax.experimental.pallas import tpu_sc as plsc

N = 10000
E = 320000
D = 128
G = 64

NC = 2
NS = 16
CHUNK = 128
CPT0 = 128
CPT1 = 32
EPT0 = CPT0 * CHUNK
EPT1 = CPT1 * CHUNK
E_PAD = NS * (EPT0 + EPT1)
HCPT = 32
NST0 = CPT0 // HCPT
NST1 = CPT1 // HCPT
NG = HCPT // 2
N_PAD = 10112
ROWS_PER_SID = N_PAD // NS


def _sc_scatter_kernel():
    mesh = plsc.VectorSubcoreMesh(
        core_axis_name="c", subcore_axis_name="s", num_cores=NC, num_subcores=NS
    )

    @functools.partial(
        pl.kernel,
        mesh=mesh,
        out_type=jax.ShapeDtypeStruct((NC, N_PAD, D), jnp.float32),
        scratch_types=[
            pltpu.VMEM((HCPT, CHUNK), jnp.int32),
            pltpu.VMEM((HCPT, CHUNK), jnp.int32),
            pltpu.VMEM((CHUNK, D), jnp.float32),
            pltpu.VMEM((CHUNK, D), jnp.float32),
            pltpu.VMEM_SHARED((N_PAD, D), jnp.float32),
            pltpu.SemaphoreType.DMA,
            pltpu.SemaphoreType.DMA,
        ],
    )
    def body(x_hbm, src0_hbm, dst0_hbm, src1_hbm, dst1_hbm, zeros_hbm, out_hbm,
             src_v, dst_v, rows_p, rows_q, agg_sh, sem_gp, sem_gq):
        cid = lax.axis_index("c")
        sid = lax.axis_index("s")

        r0 = sid * ROWS_PER_SID
        with jax.named_scope("zinit"):
            pltpu.sync_copy(
                zeros_hbm.at[pl.ds(r0, ROWS_PER_SID)], agg_sh.at[pl.ds(r0, ROWS_PER_SID)]
            )
            plsc.subcore_barrier()

        def wait_rows(buf, sem):
            pltpu.make_async_copy(zeros_hbm.at[pl.ds(0, CHUNK)], buf, sem).wait()

        def run_stage(src_hbm, dst_hbm, h):
            pltpu.sync_copy(src_hbm.at[sid, pl.ds(h * HCPT, HCPT)], src_v)
            pltpu.sync_copy(dst_hbm.at[sid, pl.ds(h * HCPT, HCPT)], dst_v)
            pltpu.async_copy(x_hbm.at[src_v.at[0]], rows_p, sem_gp)

            def pipe_body(i, carry):
                c0 = 2 * i
                c1 = c0 + 1
                wait_rows(rows_p, sem_gp)
                pltpu.async_copy(x_hbm.at[src_v.at[c1]], rows_q, sem_gq)
                pltpu.sync_copy(rows_p, agg_sh.at[dst_v.at[c0]], add=True)
                wait_rows(rows_q, sem_gq)

                @pl.when(i < NG - 1)
                def _next_gp():
                    pltpu.async_copy(x_hbm.at[src_v.at[c0 + 2]], rows_p, sem_gp)

                pltpu.sync_copy(rows_q, agg_sh.at[dst_v.at[c1]], add=True)
                return carry

            lax.fori_loop(0, NG, pipe_body, 0)

        with jax.named_scope("edges"):
            @pl.when(cid == 0)
            def _core0():
                for h in range(NST0):
                    run_stage(src0_hbm, dst0_hbm, h)

            @pl.when(cid == 1)
            def _core1():
                for h in range(NST1):
                    run_stage(src1_hbm, dst1_hbm, h)

            plsc.subcore_barrier()

        with jax.named_scope("outcpy"):
            pltpu.sync_copy(
                agg_sh.at[pl.ds(r0, ROWS_PER_SID)], out_hbm.at[cid, pl.ds(r0, ROWS_PER_SID)]
            )

    return body


_SC_SCATTER_CACHE = []


def _get_sc_scatter():
    if not _SC_SCATTER_CACHE:
        _SC_SCATTER_CACHE.append(_sc_scatter_kernel())
    return _SC_SCATTER_CACHE[0]


BLK = 400
NBLK = N // BLK


def _tc_dense_body(p0_ref, p1_ref, x_ref, gid_ref, wmsg_ref, wupd_ref, wcls_ref,
                   b_ref, out_ref, gacc, wf):
    i = pl.program_id(0)

    @pl.when(i == 0)
    def _init():
        gacc[...] = jnp.zeros((G, D), jnp.float32)
        wf[...] = jnp.dot(
            wmsg_ref[...], wupd_ref[...],
            preferred_element_type=jnp.float32, precision=lax.Precision.HIGHEST,
        )

    aggsum = p0_ref[...] + p1_ref[...]
    h = jnp.dot(aggsum, wf[...], preferred_element_type=jnp.float32,
                precision=lax.Precision.HIGHEST) + x_ref[...]
    h = jnp.maximum(h, 0.0)

    ids = gid_ref[0, 0, :]
    onehot = (lax.broadcasted_iota(jnp.int32, (G, BLK), 0) == ids[None, :]
              ).astype(jnp.float32)
    gacc[...] += jnp.dot(onehot, h, preferred_element_type=jnp.float32,
                         precision=lax.Precision.HIGHEST)

    @pl.when(i == NBLK - 1)
    def _fin():
        logits = jnp.dot(gacc[...], wcls_ref[...], preferred_element_type=jnp.float32,
                         precision=lax.Precision.HIGHEST) + b_ref[0, 0]
        out_ref[...] = jax.nn.sigmoid(logits)


_TC_DENSE = pl.pallas_call(
    _tc_dense_body,
    grid=(NBLK,),
    in_specs=[
        pl.BlockSpec((BLK, D), lambda i: (i, 0)),
        pl.BlockSpec((BLK, D), lambda i: (i, 0)),
        pl.BlockSpec((BLK, D), lambda i: (i, 0)),
        pl.BlockSpec((1, 1, BLK), lambda i: (i, 0, 0)),
        pl.BlockSpec((D, D), lambda i: (0, 0)),
        pl.BlockSpec((D, D), lambda i: (0, 0)),
        pl.BlockSpec((D, D), lambda i: (0, 0)),
        pl.BlockSpec(memory_space=pltpu.SMEM),
    ],
    out_specs=pl.BlockSpec((G, D), lambda i: (0, 0)),
    out_shape=jax.ShapeDtypeStruct((G, D), jnp.float32),
    scratch_shapes=[
        pltpu.VMEM((G, D), jnp.float32),
        pltpu.VMEM((D, D), jnp.float32),
    ],
)


def kernel(x, edge_index, graph_ids, W_msg, W_upd, W_cls, b_cls):
    src = edge_index[0]
    dst = edge_index[1]
    pad = E_PAD - E
    src_p = jnp.concatenate([src, jnp.zeros((pad,), jnp.int32)])
    dst_pad_rows = N + jnp.arange(pad, dtype=jnp.int32) % (N_PAD - N)
    dst_p = jnp.concatenate([dst, dst_pad_rows])
    split = NS * EPT0
    src_r0 = src_p[:split].reshape(NS, CPT0, CHUNK)
    dst_r0 = dst_p[:split].reshape(NS, CPT0, CHUNK)
    src_r1 = src_p[split:].reshape(NS, CPT1, CHUNK)
    dst_r1 = dst_p[split:].reshape(NS, CPT1, CHUNK)
    zeros = jnp.zeros((N_PAD, D), jnp.float32)

    partials = _get_sc_scatter()(x, src_r0, dst_r0, src_r1, dst_r1, zeros)

    gid_r = graph_ids.reshape(NBLK, 1, BLK)
    wcls_pad = jnp.pad(W_cls, ((0, 0), (0, D - W_cls.shape[1])))
    out_full = _TC_DENSE(partials[0], partials[1], x, gid_r, W_msg, W_upd,
                         wcls_pad, b_cls.reshape(1, 1))
    return out_full[:, :1]


# speedup vs baseline: 2.5184x; 2.5184x over previous
import functools

import jax
import jax.numpy as jnp
from jax import lax
from jax.experimental import pallas as pl
from jax.experimental.pallas import tpu as pltpu
from jax.experimental.pallas import tpu_sc as plsc

N = 10000
E = 320000
D = 128
G = 64

NC = 2
NS = 16
CHUNK = 128
NTILES = NC * NS
CPT = 80
EPT = CPT * CHUNK
E_PAD = NTILES * EPT
HCPT = 40
NST = CPT // HCPT
NG = HCPT // 2
N_PAD = 10112
ROWS_PER_SID = N_PAD // NS


def _sc_scatter_kernel():
    mesh = plsc.VectorSubcoreMesh(
        core_axis_name="c", subcore_axis_name="s", num_cores=NC, num_subcores=NS
    )

    @functools.partial(
        pl.kernel,
        mesh=mesh,
        out_type=jax.ShapeDtypeStruct((NC, N_PAD, D), jnp.float32),
        scratch_types=[
            pltpu.VMEM((HCPT, CHUNK), jnp.int32),
            pltpu.VMEM((HCPT, CHUNK), jnp.int32),
            pltpu.VMEM((CHUNK, D), jnp.float32),
            pltpu.VMEM((CHUNK, D), jnp.float32),
            pltpu.VMEM_SHARED((N_PAD, D), jnp.float32),
            pltpu.SemaphoreType.DMA,
            pltpu.SemaphoreType.DMA,
        ],
    )
    def body(x_hbm, src_hbm, dst_hbm, zeros_hbm, out_hbm,
             src_v, dst_v, rows_p, rows_q, agg_sh, sem_gp, sem_gq):
        cid = lax.axis_index("c")
        sid = lax.axis_index("s")

        r0 = sid * ROWS_PER_SID
        with jax.named_scope("zinit"):
            pltpu.sync_copy(
                zeros_hbm.at[pl.ds(r0, ROWS_PER_SID)], agg_sh.at[pl.ds(r0, ROWS_PER_SID)]
            )
            plsc.subcore_barrier()

        def wait_rows(buf, sem):
            pltpu.make_async_copy(zeros_hbm.at[pl.ds(0, CHUNK)], buf, sem).wait()

        def run_stage(h):
            pltpu.sync_copy(src_hbm.at[cid, sid, pl.ds(h * HCPT, HCPT)], src_v)
            pltpu.sync_copy(dst_hbm.at[cid, sid, pl.ds(h * HCPT, HCPT)], dst_v)
            pltpu.async_copy(x_hbm.at[src_v.at[0]], rows_p, sem_gp)

            def pipe_body(i, carry):
                c0 = 2 * i
                c1 = c0 + 1
                wait_rows(rows_p, sem_gp)
                pltpu.async_copy(x_hbm.at[src_v.at[c1]], rows_q, sem_gq)
                pltpu.sync_copy(rows_p, agg_sh.at[dst_v.at[c0]], add=True)
                wait_rows(rows_q, sem_gq)

                @pl.when(i < NG - 1)
                def _next_gp():
                    pltpu.async_copy(x_hbm.at[src_v.at[c0 + 2]], rows_p, sem_gp)

                pltpu.sync_copy(rows_q, agg_sh.at[dst_v.at[c1]], add=True)
                return carry

            lax.fori_loop(0, NG, pipe_body, 0)

        with jax.named_scope("edges"):
            for h in range(NST):
                run_stage(h)
            plsc.subcore_barrier()

        with jax.named_scope("outcpy"):
            pltpu.sync_copy(
                agg_sh.at[pl.ds(r0, ROWS_PER_SID)], out_hbm.at[cid, pl.ds(r0, ROWS_PER_SID)]
            )

    return body


_SC_SCATTER_CACHE = []


def _get_sc_scatter():
    if not _SC_SCATTER_CACHE:
        _SC_SCATTER_CACHE.append(_sc_scatter_kernel())
    return _SC_SCATTER_CACHE[0]


BLK = 400
NBLK = N // BLK


def _tc_dense_body(p0_ref, p1_ref, x_ref, gid_ref, wmsg_ref, wupd_ref, wcls_ref,
                   b_ref, out_ref, gacc, wf):
    i = pl.program_id(0)

    @pl.when(i == 0)
    def _init():
        gacc[...] = jnp.zeros((G, D), jnp.float32)
        wf[...] = jnp.dot(
            wmsg_ref[...], wupd_ref[...],
            preferred_element_type=jnp.float32, precision=lax.Precision.HIGHEST,
        )

    aggsum = p0_ref[...] + p1_ref[...]
    h = jnp.dot(aggsum, wf[...], preferred_element_type=jnp.float32,
                precision=lax.Precision.HIGHEST) + x_ref[...]
    h = jnp.maximum(h, 0.0)

    ids = gid_ref[0, 0, :]
    onehot = (lax.broadcasted_iota(jnp.int32, (G, BLK), 0) == ids[None, :]
              ).astype(jnp.float32)
    gacc[...] += jnp.dot(onehot, h, preferred_element_type=jnp.float32,
                         precision=lax.Precision.HIGHEST)

    @pl.when(i == NBLK - 1)
    def _fin():
        logits = jnp.dot(gacc[...], wcls_ref[...], preferred_element_type=jnp.float32,
                         precision=lax.Precision.HIGHEST) + b_ref[0, 0]
        out_ref[...] = jax.nn.sigmoid(logits)


_TC_DENSE = pl.pallas_call(
    _tc_dense_body,
    grid=(NBLK,),
    in_specs=[
        pl.BlockSpec((BLK, D), lambda i: (i, 0)),
        pl.BlockSpec((BLK, D), lambda i: (i, 0)),
        pl.BlockSpec((BLK, D), lambda i: (i, 0)),
        pl.BlockSpec((1, 1, BLK), lambda i: (i, 0, 0)),
        pl.BlockSpec((D, D), lambda i: (0, 0)),
        pl.BlockSpec((D, D), lambda i: (0, 0)),
        pl.BlockSpec((D, D), lambda i: (0, 0)),
        pl.BlockSpec(memory_space=pltpu.SMEM),
    ],
    out_specs=pl.BlockSpec((G, D), lambda i: (0, 0)),
    out_shape=jax.ShapeDtypeStruct((G, D), jnp.float32),
    scratch_shapes=[
        pltpu.VMEM((G, D), jnp.float32),
        pltpu.VMEM((D, D), jnp.float32),
    ],
)


def kernel(x, edge_index, graph_ids, W_msg, W_upd, W_cls, b_cls):
    src = edge_index[0]
    dst = edge_index[1]
    pad = E_PAD - E
    src_pad_rows = jnp.arange(pad, dtype=jnp.int32) % N
    src_p = jnp.concatenate([src, src_pad_rows])
    dst_pad_rows = N + jnp.arange(pad, dtype=jnp.int32) % (N_PAD - N)
    dst_p = jnp.concatenate([dst, dst_pad_rows])
    src_r = src_p.reshape(NC, NS, CPT, CHUNK)
    dst_r = dst_p.reshape(NC, NS, CPT, CHUNK)
    zeros = jnp.zeros((N_PAD, D), jnp.float32)

    partials = _get_sc_scatter()(x, src_r, dst_r, zeros)

    gid_r = graph_ids.reshape(NBLK, 1, BLK)
    wcls_pad = jnp.pad(W_cls, ((0, 0), (0, D - W_cls.shape[1])))
    out_full = _TC_DENSE(partials[0], partials[1], x, gid_r, W_msg, W_upd,
                         wcls_pad, b_cls.reshape(1, 1))
    return out_full[:, :1]


# speedup vs baseline: 3.0598x; 1.2150x over previous
import functools

import jax
import jax.numpy as jnp
from jax import lax
from jax.experimental import pallas as pl
from jax.experimental.pallas import tpu as pltpu
from jax.experimental.pallas import tpu_sc as plsc

N = 10000
E = 320000
D = 128
G = 64

NC = 2
NS = 16
CHUNK = 64
NBUF = 4
NTILES = NC * NS
CPT = 160
EPT = CPT * CHUNK
E_PAD = NTILES * EPT
HCPT = 40
NST = CPT // HCPT
N_PAD = 10112
ROWS_PER_SID = N_PAD // NS


def _sc_scatter_kernel():
    mesh = plsc.VectorSubcoreMesh(
        core_axis_name="c", subcore_axis_name="s", num_cores=NC, num_subcores=NS
    )

    @functools.partial(
        pl.kernel,
        mesh=mesh,
        out_type=jax.ShapeDtypeStruct((NC, N_PAD, D), jnp.float32),
        scratch_types=[
            pltpu.VMEM((HCPT, CHUNK), jnp.int32),
            pltpu.VMEM((HCPT, CHUNK), jnp.int32),
            [pltpu.VMEM((CHUNK, D), jnp.float32)] * NBUF,
            [pltpu.SemaphoreType.DMA] * NBUF,
            pltpu.VMEM_SHARED((N_PAD, D), jnp.float32),
        ],
    )
    def body(x_hbm, src_hbm, dst_hbm, zeros_hbm, out_hbm,
             src_v, dst_v, rows, sems, agg_sh):
        cid = lax.axis_index("c")
        sid = lax.axis_index("s")

        r0 = sid * ROWS_PER_SID
        with jax.named_scope("zinit"):
            pltpu.sync_copy(
                zeros_hbm.at[pl.ds(r0, ROWS_PER_SID)], agg_sh.at[pl.ds(r0, ROWS_PER_SID)]
            )
            plsc.subcore_barrier()

        def wait_rows(buf, sem):
            pltpu.make_async_copy(zeros_hbm.at[pl.ds(0, CHUNK)], buf, sem).wait()

        def fire(b, c):
            pltpu.async_copy(x_hbm.at[src_v.at[c]], rows[b], sems[b])

        def run_stage(h):
            pltpu.sync_copy(src_hbm.at[cid, sid, pl.ds(h * HCPT, HCPT)], src_v)
            pltpu.sync_copy(dst_hbm.at[cid, sid, pl.ds(h * HCPT, HCPT)], dst_v)
            for b in range(NBUF):
                fire(b, b)

            def pipe_body(i, carry):
                for b in range(NBUF):
                    c = i * NBUF + b
                    wait_rows(rows[b], sems[b])
                    pltpu.sync_copy(rows[b], agg_sh.at[dst_v.at[c]], add=True)

                    @pl.when(c + NBUF < HCPT)
                    def _refill(b=b, c=c):
                        fire(b, c + NBUF)

                return carry

            lax.fori_loop(0, HCPT // NBUF, pipe_body, 0)

        with jax.named_scope("edges"):
            for h in range(NST):
                run_stage(h)
            plsc.subcore_barrier()

        with jax.named_scope("outcpy"):
            pltpu.sync_copy(
                agg_sh.at[pl.ds(r0, ROWS_PER_SID)], out_hbm.at[cid, pl.ds(r0, ROWS_PER_SID)]
            )

    return body


_SC_SCATTER_CACHE = []


def _get_sc_scatter():
    if not _SC_SCATTER_CACHE:
        _SC_SCATTER_CACHE.append(_sc_scatter_kernel())
    return _SC_SCATTER_CACHE[0]


BLK = 2000
NBLK = N // BLK


def _tc_dense_body(p0_ref, p1_ref, x_ref, gid_ref, wmsg_ref, wupd_ref, wcls_ref,
                   b_ref, out_ref, gacc, wf):
    i = pl.program_id(0)

    @pl.when(i == 0)
    def _init():
        gacc[...] = jnp.zeros((G, D), jnp.float32)
        wf[...] = jnp.dot(
            wmsg_ref[...], wupd_ref[...],
            preferred_element_type=jnp.float32, precision=lax.Precision.HIGHEST,
        )

    aggsum = p0_ref[...] + p1_ref[...]
    h = jnp.dot(aggsum, wf[...], preferred_element_type=jnp.float32,
                precision=lax.Precision.HIGHEST) + x_ref[...]
    h = jnp.maximum(h, 0.0)

    ids = gid_ref[0, 0, :]
    onehot = (lax.broadcasted_iota(jnp.int32, (G, BLK), 0) == ids[None, :]
              ).astype(jnp.float32)
    gacc[...] += jnp.dot(onehot, h, preferred_element_type=jnp.float32,
                         precision=lax.Precision.HIGHEST)

    @pl.when(i == NBLK - 1)
    def _fin():
        logits = jnp.dot(gacc[...], wcls_ref[...], preferred_element_type=jnp.float32,
                         precision=lax.Precision.HIGHEST) + b_ref[0, 0]
        out_ref[...] = jax.nn.sigmoid(logits)


_TC_DENSE = pl.pallas_call(
    _tc_dense_body,
    grid=(NBLK,),
    in_specs=[
        pl.BlockSpec((BLK, D), lambda i: (i, 0)),
        pl.BlockSpec((BLK, D), lambda i: (i, 0)),
        pl.BlockSpec((BLK, D), lambda i: (i, 0)),
        pl.BlockSpec((1, 1, BLK), lambda i: (i, 0, 0)),
        pl.BlockSpec((D, D), lambda i: (0, 0)),
        pl.BlockSpec((D, D), lambda i: (0, 0)),
        pl.BlockSpec((D, D), lambda i: (0, 0)),
        pl.BlockSpec(memory_space=pltpu.SMEM),
    ],
    out_specs=pl.BlockSpec((G, D), lambda i: (0, 0)),
    out_shape=jax.ShapeDtypeStruct((G, D), jnp.float32),
    scratch_shapes=[
        pltpu.VMEM((G, D), jnp.float32),
        pltpu.VMEM((D, D), jnp.float32),
    ],
)


def kernel(x, edge_index, graph_ids, W_msg, W_upd, W_cls, b_cls):
    src = edge_index[0]
    dst = edge_index[1]
    pad = E_PAD - E
    src_pad_rows = jnp.arange(pad, dtype=jnp.int32) % N
    src_p = jnp.concatenate([src, src_pad_rows])
    dst_pad_rows = N + jnp.arange(pad, dtype=jnp.int32) % (N_PAD - N)
    dst_p = jnp.concatenate([dst, dst_pad_rows])
    src_r = src_p.reshape(NC, NS, CPT, CHUNK)
    dst_r = dst_p.reshape(NC, NS, CPT, CHUNK)
    zeros = jnp.zeros((N_PAD, D), jnp.float32)

    partials = _get_sc_scatter()(x, src_r, dst_r, zeros)

    gid_r = graph_ids.reshape(NBLK, 1, BLK)
    wcls_pad = jnp.pad(W_cls, ((0, 0), (0, D - W_cls.shape[1])))
    out_full = _TC_DENSE(partials[0], partials[1], x, gid_r, W_msg, W_upd,
                         wcls_pad, b_cls.reshape(1, 1))
    return out_full[:, :1]
